# baseline (speedup 1.0000x reference)
import functools

import jax
import jax.numpy as jnp
from jax import lax
from jax.experimental import pallas as pl
from jax.experimental.pallas import tpu as pltpu
from jax.experimental.pallas import tpu_sc as plsc


def _mm_body(x_ref, w_ref, o_ref):
    o_ref[...] = jnp.dot(x_ref[...], w_ref[0],
                         preferred_element_type=jnp.float32)


def _matmul_stacked(x, W_stack, blk):
    n, d_in = x.shape
    dh = W_stack.shape[-1]
    nb = n // blk
    return pl.pallas_call(
        _mm_body,
        grid=(2, nb),
        in_specs=[
            pl.BlockSpec((blk, d_in), lambda c, i: (i, 0)),
            pl.BlockSpec((1, d_in, dh), lambda c, i: (c, 0, 0)),
        ],
        out_specs=pl.BlockSpec((blk, dh), lambda c, i: (c * nb + i, 0)),
        out_shape=jax.ShapeDtypeStruct((2 * n, dh), jnp.float32),
    )(x, W_stack)


def _make_sc_scatter(n, e, dh, nc, ns):
    eps = e // ns
    chunk = 80
    nchunk = eps // chunk
    stripe = (n // ns) // 8 * 8
    tail_base = stripe * ns
    tail = n - tail_base
    zrows = 208
    assert eps * ns == e and nchunk * chunk == eps and nchunk % 2 == 0
    assert tail % 8 == 0 and stripe % zrows == 0 and tail <= zrows
    dslices = dh // 16

    mesh = plsc.VectorSubcoreMesh(core_axis_name="c", subcore_axis_name="s")

    @functools.partial(
        pl.kernel,
        out_type=jax.ShapeDtypeStruct((n, 2 * dh), jnp.float32),
        mesh=mesh,
        compiler_params=pltpu.CompilerParams(use_tc_tiling_on_sc=False),
        scratch_types=[
            pltpu.VMEM_SHARED((n, dh), jnp.float32),
            pltpu.VMEM((zrows, dh), jnp.float32),
            pltpu.VMEM((nchunk, chunk), jnp.int32),
            pltpu.VMEM((nchunk, chunk), jnp.int32),
            pltpu.VMEM((nchunk, chunk), jnp.float32),
            pltpu.VMEM((chunk, dh), jnp.float32),
            pltpu.VMEM((chunk, dh), jnp.float32),
            pltpu.SemaphoreType.DMA,
            pltpu.SemaphoreType.DMA,
            pltpu.SemaphoreType.DMA,
            pltpu.SemaphoreType.DMA,
        ],
    )
    def sc_scatter(h_hbm, src_hbm, dst_hbm, val_hbm, out_hbm,
                   acc, zbuf, src_v, dst_v, val_v,
                   rows0, rows1, gsem0, gsem1, ssem0, ssem1):
        cid = lax.axis_index("c")
        sid = lax.axis_index("s")
        rows = (rows0, rows1)
        gsems = (gsem0, gsem1)
        ssems = (ssem0, ssem1)

        pltpu.sync_copy(src_hbm.at[cid, sid], src_v)
        pltpu.sync_copy(dst_hbm.at[sid], dst_v)
        pltpu.sync_copy(val_hbm.at[sid], val_v)

        def zero_body(i, _):
            for ds_i in range(dslices):
                zbuf[i, pl.ds(ds_i * 16, 16)] = jnp.zeros((16,), jnp.float32)
            return 0
        lax.fori_loop(0, zrows, zero_body, 0)
        base_row = pl.multiple_of(sid * stripe, 8)
        for z in range(stripe // zrows):
            pltpu.sync_copy(zbuf, acc.at[pl.ds(base_row + z * zrows, zrows)])

        @pl.when(sid == 0)
        def _zero_tail():
            pltpu.sync_copy(zbuf.at[pl.ds(0, tail)],
                            acc.at[pl.ds(tail_base, tail)])
        plsc.subcore_barrier()

        def issue_gather(k, b):
            pltpu.async_copy(h_hbm.at[src_v.at[k]], rows[b], gsems[b])

        def wait_gather(k, b):
            pltpu.make_async_copy(h_hbm.at[src_v.at[k]],
                                  rows[b], gsems[b]).wait()

        def issue_scatter(k, b):
            pltpu.async_copy(rows[b], acc.at[dst_v.at[k]], ssems[b],
                             add=True)

        def wait_scatter(k, b):
            pltpu.make_async_copy(rows[b], acc.at[dst_v.at[k]],
                                  ssems[b]).wait()

        def scale(k, rbuf):
            for g in range(chunk // 16):
                v_grp = val_v[k, pl.ds(g * 16, 16)]
                for j in range(16):
                    sp = jnp.broadcast_to(v_grp[j], (16,))
                    ei = g * 16 + j
                    for ds_i in range(dslices):
                        seg = rbuf[ei, pl.ds(ds_i * 16, 16)]
                        rbuf[ei, pl.ds(ds_i * 16, 16)] = seg * sp

        issue_gather(0, 0)
        wait_gather(0, 0)
        scale(0, rows0)
        issue_gather(1, 1)
        issue_scatter(0, 0)

        def pair_body(t, _):
            k1 = 2 * t + 1
            wait_gather(k1, 1)
            scale(k1, rows1)
            wait_scatter(k1 - 1, 0)
            issue_gather(k1 + 1, 0)
            issue_scatter(k1, 1)

            k2 = 2 * t + 2
            wait_gather(k2, 0)
            scale(k2, rows0)
            wait_scatter(k2 - 1, 1)

            @pl.when(k2 + 1 < nchunk)
            def _g():
                issue_gather(k2 + 1, 1)
            issue_scatter(k2, 0)
            return 0
        lax.fori_loop(0, (nchunk - 2) // 2, pair_body, 0)

        kl = nchunk - 1
        wait_gather(kl, 1)
        scale(kl, rows1)
        wait_scatter(kl - 1, 0)
        issue_scatter(kl, 1)
        wait_scatter(kl, 1)

        plsc.subcore_barrier()
        col = pl.multiple_of(cid * dh, 8)

        def drain(rbase, nrows):
            pltpu.sync_copy(acc.at[pl.ds(rbase, nrows)],
                            zbuf.at[pl.ds(0, nrows)])

            def relu_body(i, _):
                for ds_i in range(dslices):
                    seg = zbuf[i, pl.ds(ds_i * 16, 16)]
                    zbuf[i, pl.ds(ds_i * 16, 16)] = jnp.maximum(seg, 0.0)
                return 0
            lax.fori_loop(0, nrows, relu_body, 0)
            pltpu.sync_copy(zbuf.at[pl.ds(0, nrows)],
                            out_hbm.at[pl.ds(rbase, nrows), pl.ds(col, dh)])

        for z in range(stripe // zrows):
            drain(pl.multiple_of(base_row + z * zrows, 8), zrows)

        @pl.when(sid == 0)
        def _drain_tail():
            drain(tail_base, tail)

    return sc_scatter


def kernel(x, adj_indices, adj_values, W):
    n, d_in = x.shape
    d_out = W.shape[1]
    e = adj_values.shape[0]
    dh = d_out // 2

    info = plsc.get_sparse_core_info()
    nc, ns = info.num_cores, info.num_subcores

    W_stack = jnp.stack([W[:, :dh], W[:, dh:]])
    h = _matmul_stacked(x, W_stack, blk=2000)

    eps = e // ns
    chunk = 80
    nchunk = eps // chunk
    dst = adj_indices[0].reshape(ns, nchunk, chunk)
    src = adj_indices[1].reshape(ns, nchunk, chunk)
    src_off = jnp.stack([src, src + n])
    vals = adj_values.reshape(ns, nchunk, chunk)
    sc = _make_sc_scatter(n, e, dh, nc, ns)
    return sc(h, src_off, dst, vals)


# speedup vs baseline: 1.6527x; 1.6527x over previous
import functools

import jax
import jax.numpy as jnp
from jax import lax
from jax.experimental import pallas as pl
from jax.experimental.pallas import tpu as pltpu
from jax.experimental.pallas import tpu_sc as plsc


def _mm_body(x_ref, w_ref, o_ref):
    o_ref[...] = jnp.dot(x_ref[...], w_ref[0],
                         preferred_element_type=jnp.float32)


def _matmul_stacked(x, W_stack, blk):
    n, d_in = x.shape
    dh = W_stack.shape[-1]
    nb = n // blk
    return pl.pallas_call(
        _mm_body,
        grid=(2, nb),
        in_specs=[
            pl.BlockSpec((blk, d_in), lambda c, i: (i, 0)),
            pl.BlockSpec((1, d_in, dh), lambda c, i: (c, 0, 0)),
        ],
        out_specs=pl.BlockSpec((blk, dh), lambda c, i: (c * nb + i, 0)),
        out_shape=jax.ShapeDtypeStruct((2 * n, dh), jnp.float32),
    )(x, W_stack)


def _make_sc_scatter(n, e, dh, nc, ns):
    eps = e // ns
    chunk = 80
    nchunk = eps // chunk
    stripe = (n // ns) // 8 * 8
    tail_base = stripe * ns
    tail = n - tail_base
    zrows = 208
    assert eps * ns == e and nchunk * chunk == eps and nchunk % 2 == 0
    assert tail % 8 == 0 and stripe % zrows == 0 and tail <= zrows
    dslices = dh // 16

    mesh = plsc.VectorSubcoreMesh(core_axis_name="c", subcore_axis_name="s")

    @functools.partial(
        pl.kernel,
        out_type=jax.ShapeDtypeStruct((n, 2 * dh), jnp.float32),
        mesh=mesh,
        compiler_params=pltpu.CompilerParams(use_tc_tiling_on_sc=False),
        scratch_types=[
            pltpu.VMEM_SHARED((n, dh), jnp.float32),
            pltpu.VMEM((zrows, dh), jnp.float32),
            pltpu.VMEM((nchunk, chunk), jnp.int32),
            pltpu.VMEM((nchunk, chunk), jnp.int32),
            pltpu.VMEM((nchunk, chunk), jnp.float32),
            pltpu.VMEM((chunk, dh), jnp.float32),
            pltpu.VMEM((chunk, dh), jnp.float32),
            pltpu.VMEM((chunk, dh), jnp.float32),
            pltpu.SemaphoreType.DMA,
            pltpu.SemaphoreType.DMA,
            pltpu.SemaphoreType.DMA,
            pltpu.SemaphoreType.DMA,
            pltpu.SemaphoreType.DMA,
            pltpu.SemaphoreType.DMA,
        ],
    )
    def sc_scatter(h_hbm, src_hbm, dst_hbm, val_hbm, out_hbm,
                   acc, zbuf, src_v, dst_v, val_v,
                   rows0, rows1, rows2, gsem0, gsem1, gsem2,
                   ssem0, ssem1, ssem2):
        cid = lax.axis_index("c")
        sid = lax.axis_index("s")
        rows = (rows0, rows1, rows2)
        gsems = (gsem0, gsem1, gsem2)
        ssems = (ssem0, ssem1, ssem2)

        pltpu.sync_copy(src_hbm.at[sid], src_v)
        pltpu.sync_copy(dst_hbm.at[sid], dst_v)
        pltpu.sync_copy(val_hbm.at[sid], val_v)

        @pl.when(cid == 1)
        def _offset_src():
            off = jnp.full((16,), n, jnp.int32)

            def add_body(k, _):
                for g in range(chunk // 16):
                    s = src_v[k, pl.ds(g * 16, 16)]
                    src_v[k, pl.ds(g * 16, 16)] = s + off
                return 0
            lax.fori_loop(0, nchunk, add_body, 0)

        def zero_body(i, _):
            for ds_i in range(dslices):
                zbuf[i, pl.ds(ds_i * 16, 16)] = jnp.zeros((16,), jnp.float32)
            return 0
        lax.fori_loop(0, zrows, zero_body, 0)
        base_row = pl.multiple_of(sid * stripe, 8)
        for z in range(stripe // zrows):
            pltpu.sync_copy(zbuf, acc.at[pl.ds(base_row + z * zrows, zrows)])

        @pl.when(sid == 0)
        def _zero_tail():
            pltpu.sync_copy(zbuf.at[pl.ds(0, tail)],
                            acc.at[pl.ds(tail_base, tail)])
        plsc.subcore_barrier()

        def issue_gather(k, b):
            pltpu.async_copy(h_hbm.at[src_v.at[k]], rows[b], gsems[b])

        def wait_gather(k, b):
            pltpu.make_async_copy(h_hbm.at[src_v.at[k]],
                                  rows[b], gsems[b]).wait()

        def issue_scatter(k, b):
            pltpu.async_copy(rows[b], acc.at[dst_v.at[k]], ssems[b],
                             add=True)

        def wait_scatter(k, b):
            pltpu.make_async_copy(rows[b], acc.at[dst_v.at[k]],
                                  ssems[b]).wait()

        def scale(k, rbuf):
            for g in range(chunk // 16):
                v_grp = val_v[k, pl.ds(g * 16, 16)]
                for j in range(16):
                    sp = jnp.broadcast_to(v_grp[j], (16,))
                    ei = g * 16 + j
                    for ds_i in range(dslices):
                        seg = rbuf[ei, pl.ds(ds_i * 16, 16)]
                        rbuf[ei, pl.ds(ds_i * 16, 16)] = seg * sp

        assert (nchunk - 4) % 3 == 0

        def steady(k, b, bm1):
            wait_gather(k, b)
            scale(k, rows[b])
            wait_scatter(k - 1, bm1)
            issue_gather(k + 2, bm1)
            issue_scatter(k, b)

        issue_gather(0, 0)
        issue_gather(1, 1)
        wait_gather(0, 0)
        scale(0, rows0)
        issue_gather(2, 2)
        issue_scatter(0, 0)

        def tri_body(t, _):
            steady(3 * t + 1, 1, 0)
            steady(3 * t + 2, 2, 1)
            steady(3 * t + 3, 0, 2)
            return 0
        lax.fori_loop(0, (nchunk - 4) // 3, tri_body, 0)

        k = nchunk - 3
        b = (nchunk - 3) % 3
        wait_gather(k, b)
        scale(k, rows[b])
        wait_scatter(k - 1, (k - 1) % 3)
        issue_gather(k + 2, (k - 1) % 3)
        issue_scatter(k, b)

        k = nchunk - 2
        b = (nchunk - 2) % 3
        wait_gather(k, b)
        scale(k, rows[b])
        wait_scatter(k - 1, (k - 1) % 3)
        issue_scatter(k, b)

        k = nchunk - 1
        b = (nchunk - 1) % 3
        wait_gather(k, b)
        scale(k, rows[b])
        wait_scatter(k - 1, (k - 1) % 3)
        issue_scatter(k, b)
        wait_scatter(k, b)

        plsc.subcore_barrier()
        col = pl.multiple_of(cid * dh, 8)

        def drain(rbase, nrows):
            pltpu.sync_copy(acc.at[pl.ds(rbase, nrows)],
                            zbuf.at[pl.ds(0, nrows)])

            def relu_body(i, _):
                for ds_i in range(dslices):
                    seg = zbuf[i, pl.ds(ds_i * 16, 16)]
                    zbuf[i, pl.ds(ds_i * 16, 16)] = jnp.maximum(seg, 0.0)
                return 0
            lax.fori_loop(0, nrows, relu_body, 0)
            pltpu.sync_copy(zbuf.at[pl.ds(0, nrows)],
                            out_hbm.at[pl.ds(rbase, nrows), pl.ds(col, dh)])

        for z in range(stripe // zrows):
            drain(pl.multiple_of(base_row + z * zrows, 8), zrows)

        @pl.when(sid == 0)
        def _drain_tail():
            drain(tail_base, tail)

    return sc_scatter


def kernel(x, adj_indices, adj_values, W):
    n, d_in = x.shape
    d_out = W.shape[1]
    e = adj_values.shape[0]
    dh = d_out // 2

    info = plsc.get_sparse_core_info()
    nc, ns = info.num_cores, info.num_subcores

    W_stack = jnp.stack([W[:, :dh], W[:, dh:]])
    h = _matmul_stacked(x, W_stack, blk=2000)

    eps = e // ns
    chunk = 80
    nchunk = eps // chunk
    dst = adj_indices[0].reshape(ns, nchunk, chunk)
    src = adj_indices[1].reshape(ns, nchunk, chunk)
    vals = adj_values.reshape(ns, nchunk, chunk)
    sc = _make_sc_scatter(n, e, dh, nc, ns)
    return sc(h, src, dst, vals)
